# baseline (speedup 1.0000x reference)
import functools

import jax
import jax.numpy as jnp
from jax import lax
from jax.experimental import pallas as pl
from jax.experimental.pallas import tpu as pltpu
from jax.experimental.pallas import tpu_sc as plsc


EMBED = 64
PE_LEN = 256
LANES = 16


def _fuse_tables(token_table, language_table, bounds):
    V, E = token_table.shape
    BLK = 25000
    grid = V // BLK

    def body(b_ref, lang_ref, tok_ref, out_ref):
        i = pl.program_id(0)
        rows = tok_ref[...]
        rid = lax.broadcasted_iota(jnp.int32, (BLK, 1), 0) + i * BLK
        lang = lang_ref[...]
        for off, l in ((0, 3), (2, 2), (4, 1)):
            lo = b_ref[off]
            hi = b_ref[off + 1]
            m = (rid >= lo) & (rid <= hi)
            rows = rows + jnp.where(m, lang[l][None, :], 0.0)
        out_ref[...] = rows

    return pl.pallas_call(
        body,
        grid=(grid,),
        in_specs=[
            pl.BlockSpec(memory_space=pltpu.SMEM),
            pl.BlockSpec((4, E), lambda i: (0, 0)),
            pl.BlockSpec((BLK, E), lambda i: (i, 0)),
        ],
        out_specs=pl.BlockSpec((BLK, E), lambda i: (i, 0)),
        out_shape=jax.ShapeDtypeStruct((V, E), jnp.float32),
    )(bounds, language_table, token_table)


def _sc_gather(fused, seq2d, pe):
    info = plsc.get_sparse_core_info()
    NC, NS = info.num_cores, info.num_subcores
    NW = NC * NS
    TOTAL = seq2d.shape[0] * seq2d.shape[1]
    PER_W = TOTAL // NW
    CHUNK = 512
    IDXW = 128
    NSUB = CHUNK // IDXW
    NCH = PER_W // CHUNK
    mesh = plsc.VectorSubcoreMesh(core_axis_name="c", subcore_axis_name="s")

    @functools.partial(
        pl.kernel,
        mesh=mesh,
        out_type=jax.ShapeDtypeStruct((TOTAL, EMBED), jnp.float32),
        compiler_params=pltpu.CompilerParams(use_tc_tiling_on_sc=False),
        scratch_types=[
            pltpu.VMEM((NSUB, IDXW), jnp.int32),
            pltpu.VMEM((CHUNK, EMBED), jnp.float32),
            pltpu.VMEM((PE_LEN, EMBED), jnp.float32),
            pltpu.SemaphoreType.DMA,
        ],
    )
    def k(fused_hbm, seq_hbm, pe_hbm, out_hbm, idx_v, rows_v, pe_v, sem):
        wid = lax.axis_index("s") * NC + lax.axis_index("c")
        pltpu.sync_copy(pe_hbm, pe_v)

        def chunk_body(i, carry):
            base = wid * PER_W + i * CHUNK
            brow = wid * (PER_W // IDXW) + i * NSUB
            pltpu.sync_copy(seq_hbm.at[pl.ds(brow, NSUB)], idx_v)
            descs = [
                pltpu.async_copy(
                    fused_hbm.at[idx_v.at[j]],
                    rows_v.at[pl.ds(j * IDXW, IDXW)],
                    sem,
                )
                for j in range(NSUB)
            ]
            for d in descs:
                d.wait()

            @plsc.parallel_loop(0, PE_LEN, unroll=8)
            def pe_body(p):
                for h in range(CHUNK // PE_LEN):
                    r = p + h * PE_LEN
                    for c in range(EMBED // LANES):
                        sl = pl.ds(c * LANES, LANES)
                        rows_v[r, sl] = rows_v[r, sl] + pe_v[p, sl]
            pltpu.sync_copy(rows_v, out_hbm.at[pl.ds(base, CHUNK)])
            return carry

        lax.fori_loop(0, NCH, chunk_body, 0)

    return k(fused, seq2d, pe)


def kernel(sequence, token_table, language_table, pe, ms_vocab, eng_vocab, chi_vocab):
    B, S = sequence.shape
    bounds = jnp.stack(
        [
            ms_vocab[0].astype(jnp.int32),
            ms_vocab[-1].astype(jnp.int32),
            eng_vocab[0].astype(jnp.int32),
            eng_vocab[-1].astype(jnp.int32),
            chi_vocab[0].astype(jnp.int32),
            chi_vocab[-1].astype(jnp.int32),
        ]
    )
    fused = _fuse_tables(token_table, language_table, bounds)
    seq2d = sequence.astype(jnp.int32).reshape(-1, 128)
    out = _sc_gather(fused, seq2d, pe)
    return out.reshape(B, S, EMBED)


# speedup vs baseline: 1.1261x; 1.1261x over previous
import functools

import jax
import jax.numpy as jnp
from jax import lax
from jax.experimental import pallas as pl
from jax.experimental.pallas import tpu as pltpu
from jax.experimental.pallas import tpu_sc as plsc


EMBED = 64
PE_LEN = 256
LANES = 16


def _fuse_tables(token_table, language_table, bounds):
    V, E = token_table.shape
    BLK = 25000
    grid = V // BLK

    def body(b_ref, lang_ref, tok_ref, out_ref):
        i = pl.program_id(0)
        rows = tok_ref[...]
        rid = lax.broadcasted_iota(jnp.int32, (BLK, 1), 0) + i * BLK
        lang = lang_ref[...]
        for off, l in ((0, 3), (2, 2), (4, 1)):
            lo = b_ref[off]
            hi = b_ref[off + 1]
            m = (rid >= lo) & (rid <= hi)
            rows = rows + jnp.where(m, lang[l][None, :], 0.0)
        out_ref[...] = rows

    return pl.pallas_call(
        body,
        grid=(grid,),
        in_specs=[
            pl.BlockSpec(memory_space=pltpu.SMEM),
            pl.BlockSpec((4, E), lambda i: (0, 0)),
            pl.BlockSpec((BLK, E), lambda i: (i, 0)),
        ],
        out_specs=pl.BlockSpec((BLK, E), lambda i: (i, 0)),
        out_shape=jax.ShapeDtypeStruct((V, E), jnp.float32),
    )(bounds, language_table, token_table)


def _sc_gather(fused, seq2d, pe):
    info = plsc.get_sparse_core_info()
    NC, NS = info.num_cores, info.num_subcores
    NW = NC * NS
    TOTAL = seq2d.shape[0] * seq2d.shape[1]
    PER_W = TOTAL // NW
    CHUNK = 512
    IDXW = 128
    NSUB = CHUNK // IDXW
    NCH = PER_W // CHUNK
    mesh = plsc.VectorSubcoreMesh(core_axis_name="c", subcore_axis_name="s")

    @functools.partial(
        pl.kernel,
        mesh=mesh,
        out_type=jax.ShapeDtypeStruct((TOTAL, EMBED), jnp.float32),
        compiler_params=pltpu.CompilerParams(use_tc_tiling_on_sc=False),
        scratch_types=[
            pltpu.VMEM((NSUB, IDXW), jnp.int32),
            pltpu.VMEM((NSUB, IDXW), jnp.int32),
            pltpu.VMEM((CHUNK, EMBED), jnp.float32),
            pltpu.VMEM((CHUNK, EMBED), jnp.float32),
            pltpu.VMEM((PE_LEN, EMBED), jnp.float32),
            pltpu.SemaphoreType.DMA,
            pltpu.SemaphoreType.DMA,
        ],
    )
    def k(fused_hbm, seq_hbm, pe_hbm, out_hbm, idx0, idx1, rows0, rows1, pe_v, sem0, sem1):
        wid = lax.axis_index("s") * NC + lax.axis_index("c")
        pltpu.sync_copy(pe_hbm, pe_v)

        def issue(ci, idx_v, rows_v, sem):
            brow = wid * (PER_W // IDXW) + ci * NSUB
            pltpu.sync_copy(seq_hbm.at[pl.ds(brow, NSUB)], idx_v)
            for j in range(NSUB):
                pltpu.async_copy(
                    fused_hbm.at[idx_v.at[j]],
                    rows_v.at[pl.ds(j * IDXW, IDXW)],
                    sem,
                )

        def drain(rows_v, sem):
            pltpu.make_async_copy(fused_hbm.at[pl.ds(0, CHUNK)], rows_v, sem).wait()

        def process(ci, rows_v):
            @plsc.parallel_loop(0, PE_LEN, unroll=8)
            def pe_body(p):
                for h in range(CHUNK // PE_LEN):
                    r = p + h * PE_LEN
                    for c in range(EMBED // LANES):
                        sl = pl.ds(c * LANES, LANES)
                        rows_v[r, sl] = rows_v[r, sl] + pe_v[p, sl]
            base = wid * PER_W + ci * CHUNK
            pltpu.sync_copy(rows_v, out_hbm.at[pl.ds(base, CHUNK)])

        issue(0, idx0, rows0, sem0)

        def pair_body(i2, carry):
            c0 = i2 * 2
            issue(c0 + 1, idx1, rows1, sem1)
            drain(rows0, sem0)
            process(c0, rows0)
            c2 = jnp.minimum(c0 + 2, NCH - 1)
            issue(c2, idx0, rows0, sem0)
            drain(rows1, sem1)
            process(c0 + 1, rows1)
            return carry

        lax.fori_loop(0, NCH // 2, pair_body, 0)
        drain(rows0, sem0)

    return k(fused, seq2d, pe)


def kernel(sequence, token_table, language_table, pe, ms_vocab, eng_vocab, chi_vocab):
    B, S = sequence.shape
    bounds = jnp.stack(
        [
            ms_vocab[0].astype(jnp.int32),
            ms_vocab[-1].astype(jnp.int32),
            eng_vocab[0].astype(jnp.int32),
            eng_vocab[-1].astype(jnp.int32),
            chi_vocab[0].astype(jnp.int32),
            chi_vocab[-1].astype(jnp.int32),
        ]
    )
    fused = _fuse_tables(token_table, language_table, bounds)
    seq2d = sequence.astype(jnp.int32).reshape(-1, 128)
    out = _sc_gather(fused, seq2d, pe)
    return out.reshape(B, S, EMBED)
